# baseline (speedup 1.0000x reference)
import functools

import jax
import jax.numpy as jnp
from jax import lax
from jax.experimental import pallas as pl
from jax.experimental.pallas import tpu as pltpu
from jax.experimental.pallas import tpu_sc as plsc

NC = 2
NS = 16
NW = NC * NS
L = 16

_SC_PARAMS = pltpu.CompilerParams(
    needs_layout_passes=False, use_tc_tiling_on_sc=False
)


def _degree_call(src_idx, dst_idx, n, e):
    ec = e // NW
    deg_b = 2000
    mesh = plsc.VectorSubcoreMesh(core_axis_name="c", subcore_axis_name="s")

    @functools.partial(
        pl.kernel,
        out_type=jax.ShapeDtypeStruct((NW, 2, n), jnp.float32),
        mesh=mesh,
        compiler_params=_SC_PARAMS,
        scratch_types=[
            pltpu.VMEM((deg_b,), jnp.int32),
            pltpu.VMEM((n,), jnp.float32),
            pltpu.VMEM((n,), jnp.float32),
        ],
    )
    def deg_kernel(src_hbm, dst_hbm, out_hbm, idx_v, dgo_v, dgi_v):
        c = lax.axis_index("c")
        s = lax.axis_index("s")
        w = c * NS + s
        zero = jnp.zeros((L,), jnp.float32)
        ones = jnp.ones((L,), jnp.float32)

        @pl.loop(0, n // L)
        def _zero(i):
            dgo_v[pl.ds(i * L, L)] = zero
            dgi_v[pl.ds(i * L, L)] = zero

        @pl.loop(0, ec // deg_b)
        def _chunk(k):
            base = w * ec + k * deg_b
            pltpu.sync_copy(src_hbm.at[pl.ds(base, deg_b)], idx_v)

            @pl.loop(0, deg_b // L)
            def _src(j):
                iv = idx_v[pl.ds(j * L, L)]
                plsc.addupdate_scatter(dgo_v, [iv], ones)

            pltpu.sync_copy(dst_hbm.at[pl.ds(base, deg_b)], idx_v)

            @pl.loop(0, deg_b // L)
            def _dst(j):
                iv = idx_v[pl.ds(j * L, L)]
                plsc.addupdate_scatter(dgi_v, [iv], ones)

        pltpu.sync_copy(dgo_v, out_hbm.at[w, 0])
        pltpu.sync_copy(dgi_v, out_hbm.at[w, 1])

    return deg_kernel(src_idx, dst_idx)


def _node_tables_call(x, w_src, b_src, w_dst, b_dst, degp):
    n, d = x.shape
    h = d // 2
    nb = n
    grid = 1

    def body(x_ref, ws_ref, bs_ref, wd_ref, bd_ref, degp_ref, tsrc_ref, tdst_ref):
        xv = x_ref[...]
        es = jnp.dot(xv, ws_ref[...], preferred_element_type=jnp.float32) + bs_ref[...]
        deg_o = jnp.sum(degp_ref[:, 0, :], axis=0)
        inv_o = lax.rsqrt(jnp.maximum(deg_o, 1.0))
        fs = xv * inv_o[:, None]
        tsrc_ref[0, :, :h] = es[:, :h]
        tsrc_ref[0, :, h:] = fs[:, :h]
        tsrc_ref[1, :, :h] = es[:, h:]
        tsrc_ref[1, :, h:] = fs[:, h:]
        ed = jnp.dot(xv, wd_ref[...], preferred_element_type=jnp.float32) + bd_ref[...]
        tdst_ref[0] = ed[:, :h]
        tdst_ref[1] = ed[:, h:]

    return pl.pallas_call(
        body,
        grid=(grid,),
        in_specs=[
            pl.BlockSpec((nb, d), lambda i: (i, 0)),
            pl.BlockSpec((d, d), lambda i: (0, 0)),
            pl.BlockSpec((1, d), lambda i: (0, 0)),
            pl.BlockSpec((d, d), lambda i: (0, 0)),
            pl.BlockSpec((1, d), lambda i: (0, 0)),
            pl.BlockSpec((NW, 2, n), lambda i: (0, 0, 0)),
        ],
        out_specs=[
            pl.BlockSpec((NC, nb, d), lambda i: (0, i, 0)),
            pl.BlockSpec((NC, nb, h), lambda i: (0, i, 0)),
        ],
        out_shape=[
            jax.ShapeDtypeStruct((NC, n, d), jnp.float32),
            jax.ShapeDtypeStruct((NC, n, h), jnp.float32),
        ],
    )(x, w_src, b_src, w_dst, b_dst, degp)


def _edge_lin_call(edge_feats, w_edge, b_edge):
    e, d = edge_feats.shape
    h = d // 2
    be = 2000
    grid = e // be

    def body(ef_ref, we_ref, bb_ref, out_ref):
        lin = (
            jnp.dot(ef_ref[...], we_ref[...], preferred_element_type=jnp.float32)
            + bb_ref[...]
        )
        out_ref[0] = lin[:, :h]
        out_ref[1] = lin[:, h:]

    return pl.pallas_call(
        body,
        grid=(grid,),
        in_specs=[
            pl.BlockSpec((be, d), lambda i: (i, 0)),
            pl.BlockSpec((d, d), lambda i: (0, 0)),
            pl.BlockSpec((1, d), lambda i: (0, 0)),
        ],
        out_specs=pl.BlockSpec((NC, be, h), lambda i: (0, i, 0)),
        out_shape=jax.ShapeDtypeStruct((NC, e, h), jnp.float32),
    )(edge_feats, w_edge, b_edge)


def _edge_gather_scatter_call(src_idx, dst_idx, tsrc, tdst, el, n, e, d):
    h = d // 2
    ec = e // NS
    b = 80
    nchunk = ec // b
    nps = 624
    tail = n - NS * nps
    zr = 208
    mesh = plsc.VectorSubcoreMesh(core_axis_name="c", subcore_axis_name="s")

    @functools.partial(
        pl.kernel,
        out_type=(
            jax.ShapeDtypeStruct((e, d), jnp.float32),
            jax.ShapeDtypeStruct((NC, n, h), jnp.float32),
        ),
        mesh=mesh,
        compiler_params=_SC_PARAMS,
        scratch_types=[
            [pltpu.VMEM((1, b), jnp.int32)] * 2,
            [pltpu.VMEM((1, b), jnp.int32)] * 2,
            [pltpu.VMEM((b, d), jnp.float32)] * 2,
            [pltpu.VMEM((b, h), jnp.float32)] * 2,
            [pltpu.VMEM((b, h), jnp.float32)] * 2,
            [pltpu.VMEM((b, h), jnp.float32)] * 2,
            [pltpu.VMEM((b, h), jnp.float32)] * 2,
            pltpu.VMEM((zr, h), jnp.float32),
            pltpu.VMEM_SHARED((n, h), jnp.float32),
            [pltpu.SemaphoreType.DMA] * 2,
            [pltpu.SemaphoreType.DMA] * 2,
            [pltpu.SemaphoreType.DMA] * 2,
        ],
    )
    def edge_kernel(src_hbm, dst_hbm, tsrc_hbm, tdst_hbm, el_hbm, m_hbm, acc_hbm,
                    idx_sv, idx_dv, gsrc, gdst, elb, m_buf, msg, zbuf, acc_sh,
                    sem_g, sem_m, sem_s):
        c = lax.axis_index("c")
        s = lax.axis_index("s")
        zero = jnp.zeros((L,), jnp.float32)

        @pl.loop(0, zr)
        def _zfill(r):
            for cc in range(h // L):
                zbuf[r, pl.ds(cc * L, L)] = zero

        for j in range(nps // zr):
            pltpu.sync_copy(zbuf, acc_sh.at[pl.ds(s * nps + j * zr, zr)])

        @pl.when(s == NS - 1)
        def _ztail():
            pltpu.sync_copy(zbuf.at[pl.ds(0, tail)], acc_sh.at[pl.ds(NS * nps, tail)])

        plsc.subcore_barrier()

        def issue_gathers(k, bb):
            base = s * ec + k * b
            g = s * nchunk + k
            pltpu.sync_copy(src_hbm.at[pl.ds(g, 1)], idx_sv[bb])
            pltpu.sync_copy(dst_hbm.at[pl.ds(g, 1)], idx_dv[bb])
            pltpu.async_copy(tsrc_hbm.at[c].at[idx_sv[bb].at[0]], gsrc[bb], sem_g[bb])
            pltpu.async_copy(tdst_hbm.at[c].at[idx_dv[bb].at[0]], gdst[bb], sem_g[bb])
            pltpu.async_copy(el_hbm.at[c, pl.ds(base, b)], elb[bb], sem_g[bb])

        def wait_gathers(bb):
            pltpu.make_async_copy(
                tsrc_hbm.at[c].at[idx_sv[bb].at[0]], gsrc[bb], sem_g[bb]).wait()
            pltpu.make_async_copy(
                tdst_hbm.at[c].at[idx_dv[bb].at[0]], gdst[bb], sem_g[bb]).wait()
            pltpu.make_async_copy(
                el_hbm.at[c, pl.ds(0, b)], elb[bb], sem_g[bb]).wait()

        def wait_m(bb):
            pltpu.make_async_copy(
                m_buf[bb], m_hbm.at[pl.ds(0, b), pl.ds(c * h, h)], sem_m[bb]
            ).wait()

        def wait_s(bb):
            pltpu.make_async_copy(
                el_hbm.at[c, pl.ds(0, b)], msg[bb], sem_s[bb]).wait()

        issue_gathers(0, 0)

        @pl.loop(0, nchunk // 2)
        def _chunk2(kk):
            for half in range(2):
                k = kk * 2 + half
                other = 1 - half

                if half == 0:
                    @pl.when(kk >= 1)
                    def _wprev():
                        wait_m(other)
                        wait_s(other)
                else:
                    wait_m(other)
                    wait_s(other)

                if half == 0:
                    issue_gathers(k + 1, other)
                else:
                    @pl.when(kk + 1 < nchunk // 2)
                    def _pref():
                        issue_gathers(k + 1, other)

                wait_gathers(half)

                @pl.loop(0, b, unroll=8)
                def _row(r):
                    for cc in range(h // L):
                        off = cc * L
                        es = gsrc[half][r, pl.ds(off, L)]
                        ed = gdst[half][r, pl.ds(off, L)]
                        ev = elb[half][r, pl.ds(off, L)]
                        mv = es + ed + ev
                        m_buf[half][r, pl.ds(off, L)] = mv
                        den = 1.0 + jnp.exp(-mv)
                        fs = gsrc[half][r, pl.ds(h + off, L)]
                        msg[half][r, pl.ds(off, L)] = fs / den

                base = s * ec + k * b
                pltpu.async_copy(
                    m_buf[half], m_hbm.at[pl.ds(base, b), pl.ds(c * h, h)],
                    sem_m[half],
                )
                pltpu.async_copy(
                    msg[half], acc_sh.at[idx_dv[half].at[0]], sem_s[half], add=True
                )

        wait_m(1)
        wait_s(1)
        plsc.subcore_barrier()
        pltpu.sync_copy(acc_sh.at[pl.ds(s * nps, nps)],
                        acc_hbm.at[c, pl.ds(s * nps, nps)])

        @pl.when(s == NS - 1)
        def _otail():
            pltpu.sync_copy(acc_sh.at[pl.ds(NS * nps, tail)],
                            acc_hbm.at[c, pl.ds(NS * nps, tail)])

    src2 = src_idx.reshape(e // b, b)
    dst2 = dst_idx.reshape(e // b, b)
    return edge_kernel(src2, dst2, tsrc, tdst, el)


def _final_call(x, acc, degp, weight, bias):
    n, d = x.shape
    h = d // 2
    nb = n
    grid = 1

    def body(x_ref, acc_ref, degp_ref, w_ref, bb_ref, out_ref):
        a = jnp.concatenate([acc_ref[0], acc_ref[1]], axis=1)
        r = jnp.dot(a, w_ref[...], preferred_element_type=jnp.float32)
        deg_i = jnp.sum(degp_ref[:, 1, :], axis=0)
        inv_i = lax.rsqrt(jnp.maximum(deg_i, 1.0))
        out_ref[...] = x_ref[...] + r * inv_i[:, None] + bb_ref[...]

    return pl.pallas_call(
        body,
        grid=(grid,),
        in_specs=[
            pl.BlockSpec((nb, d), lambda i: (i, 0)),
            pl.BlockSpec((NC, nb, h), lambda i: (0, i, 0)),
            pl.BlockSpec((NW, 2, n), lambda i: (0, 0, 0)),
            pl.BlockSpec((d, d), lambda i: (0, 0)),
            pl.BlockSpec((1, d), lambda i: (0, 0)),
        ],
        out_specs=pl.BlockSpec((nb, d), lambda i: (i, 0)),
        out_shape=jax.ShapeDtypeStruct((n, d), jnp.float32),
    )(x, acc, degp, weight, bias)


def kernel(node_feats, edge_index, edge_feats, W_src, b_src, W_dst, b_dst,
           W_edge, b_edge, weight, bias):
    n, d = node_feats.shape
    e = edge_index.shape[1]
    src_idx = edge_index[0]
    dst_idx = edge_index[1]
    degp = _degree_call(src_idx, dst_idx, n, e)
    tsrc, tdst = _node_tables_call(
        node_feats, W_src, b_src.reshape(1, d), W_dst, b_dst.reshape(1, d), degp
    )
    el = _edge_lin_call(edge_feats, W_edge, b_edge.reshape(1, d))
    m, acc = _edge_gather_scatter_call(src_idx, dst_idx, tsrc, tdst, el, n, e, d)
    rst = _final_call(node_feats, acc, degp, weight, bias.reshape(1, d))
    return (rst, m)


# speedup vs baseline: 2.4482x; 2.4482x over previous
import functools

import jax
import jax.numpy as jnp
from jax import lax
from jax.experimental import pallas as pl
from jax.experimental.pallas import tpu as pltpu
from jax.experimental.pallas import tpu_sc as plsc

NC = 2
NS = 16
NW = NC * NS
L = 16

_SC_PARAMS = pltpu.CompilerParams(
    needs_layout_passes=False, use_tc_tiling_on_sc=False
)


def _degree_call(src_idx, dst_idx, n, e):
    ec = e // NW
    deg_b = 2000
    mesh = plsc.VectorSubcoreMesh(core_axis_name="c", subcore_axis_name="s")

    @functools.partial(
        pl.kernel,
        out_type=jax.ShapeDtypeStruct((NW, 2, n), jnp.float32),
        mesh=mesh,
        compiler_params=_SC_PARAMS,
        scratch_types=[
            pltpu.VMEM((deg_b,), jnp.int32),
            pltpu.VMEM((n,), jnp.float32),
            pltpu.VMEM((n,), jnp.float32),
        ],
    )
    def deg_kernel(src_hbm, dst_hbm, out_hbm, idx_v, dgo_v, dgi_v):
        c = lax.axis_index("c")
        s = lax.axis_index("s")
        w = c * NS + s
        zero = jnp.zeros((L,), jnp.float32)
        ones = jnp.ones((L,), jnp.float32)

        @pl.loop(0, n // L)
        def _zero(i):
            dgo_v[pl.ds(i * L, L)] = zero
            dgi_v[pl.ds(i * L, L)] = zero

        @pl.loop(0, ec // deg_b)
        def _chunk(k):
            base = w * ec + k * deg_b
            pltpu.sync_copy(src_hbm.at[pl.ds(base, deg_b)], idx_v)

            @pl.loop(0, deg_b // L)
            def _src(j):
                iv = idx_v[pl.ds(j * L, L)]
                plsc.addupdate_scatter(dgo_v, [iv], ones)

            pltpu.sync_copy(dst_hbm.at[pl.ds(base, deg_b)], idx_v)

            @pl.loop(0, deg_b // L)
            def _dst(j):
                iv = idx_v[pl.ds(j * L, L)]
                plsc.addupdate_scatter(dgi_v, [iv], ones)

        pltpu.sync_copy(dgo_v, out_hbm.at[w, 0])
        pltpu.sync_copy(dgi_v, out_hbm.at[w, 1])

    return deg_kernel(src_idx, dst_idx)


def _node_tables_call(x, w_src, b_src, w_dst, b_dst, degp):
    n, d = x.shape
    h = d // 2
    nb = n
    grid = 1

    def body(x_ref, ws_ref, bs_ref, wd_ref, bd_ref, degp_ref, tsrc_ref, tdst_ref):
        xv = x_ref[...]
        es = jnp.dot(xv, ws_ref[...], preferred_element_type=jnp.float32) + bs_ref[...]
        deg_o = jnp.sum(degp_ref[:, 0, :], axis=0)
        inv_o = lax.rsqrt(jnp.maximum(deg_o, 1.0))
        fs = xv * inv_o[:, None]
        tsrc_ref[0, :, :h] = es[:, :h]
        tsrc_ref[0, :, h:] = fs[:, :h]
        tsrc_ref[1, :, :h] = es[:, h:]
        tsrc_ref[1, :, h:] = fs[:, h:]
        ed = jnp.dot(xv, wd_ref[...], preferred_element_type=jnp.float32) + bd_ref[...]
        tdst_ref[0] = ed[:, :h]
        tdst_ref[1] = ed[:, h:]

    return pl.pallas_call(
        body,
        grid=(grid,),
        in_specs=[
            pl.BlockSpec((nb, d), lambda i: (i, 0)),
            pl.BlockSpec((d, d), lambda i: (0, 0)),
            pl.BlockSpec((1, d), lambda i: (0, 0)),
            pl.BlockSpec((d, d), lambda i: (0, 0)),
            pl.BlockSpec((1, d), lambda i: (0, 0)),
            pl.BlockSpec((NW, 2, n), lambda i: (0, 0, 0)),
        ],
        out_specs=[
            pl.BlockSpec((NC, nb, d), lambda i: (0, i, 0)),
            pl.BlockSpec((NC, nb, h), lambda i: (0, i, 0)),
        ],
        out_shape=[
            jax.ShapeDtypeStruct((NC, n, d), jnp.float32),
            jax.ShapeDtypeStruct((NC, n, h), jnp.float32),
        ],
    )(x, w_src, b_src, w_dst, b_dst, degp)


def _edge_lin_call(edge_feats, w_edge, b_edge):
    e, d = edge_feats.shape
    h = d // 2
    be = 2000
    grid = e // be

    def body(ef_ref, we_ref, bb_ref, out_ref):
        lin = (
            jnp.dot(ef_ref[...], we_ref[...], preferred_element_type=jnp.float32)
            + bb_ref[...]
        )
        out_ref[0] = lin[:, :h]
        out_ref[1] = lin[:, h:]

    return pl.pallas_call(
        body,
        grid=(grid,),
        in_specs=[
            pl.BlockSpec((be, d), lambda i: (i, 0)),
            pl.BlockSpec((d, d), lambda i: (0, 0)),
            pl.BlockSpec((1, d), lambda i: (0, 0)),
        ],
        out_specs=pl.BlockSpec((NC, be, h), lambda i: (0, i, 0)),
        out_shape=jax.ShapeDtypeStruct((NC, e, h), jnp.float32),
    )(edge_feats, w_edge, b_edge)


def _edge_gather_scatter_call(src_idx, dst_idx, tsrc, tdst, el, n, e, d):
    h = d // 2
    ec = e // NS
    b = 80
    nchunk = ec // b
    nps = 624
    tail = n - NS * nps
    zr = 208
    mesh = plsc.VectorSubcoreMesh(core_axis_name="c", subcore_axis_name="s")

    @functools.partial(
        pl.kernel,
        out_type=(
            jax.ShapeDtypeStruct((e, d), jnp.float32),
            jax.ShapeDtypeStruct((NC, n, h), jnp.float32),
        ),
        mesh=mesh,
        compiler_params=_SC_PARAMS,
        scratch_types=[
            [pltpu.VMEM((1, b), jnp.int32)] * 2,
            [pltpu.VMEM((1, b), jnp.int32)] * 2,
            [pltpu.VMEM((b, d), jnp.float32)] * 2,
            [pltpu.VMEM((b, h), jnp.float32)] * 2,
            [pltpu.VMEM((b, h), jnp.float32)] * 2,
            [pltpu.VMEM((b, h), jnp.float32)] * 2,
            [pltpu.VMEM((b, h), jnp.float32)] * 2,
            pltpu.VMEM((zr, h), jnp.float32),
            pltpu.VMEM_SHARED((n, h), jnp.float32),
            [pltpu.SemaphoreType.DMA] * 2,
            [pltpu.SemaphoreType.DMA] * 2,
            [pltpu.SemaphoreType.DMA] * 2,
        ],
    )
    def edge_kernel(src_hbm, dst_hbm, tsrc_hbm, tdst_hbm, el_hbm, m_hbm, acc_hbm,
                    idx_sv, idx_dv, gsrc, gdst, elb, m_buf, msg, zbuf, acc_sh,
                    sem_g, sem_m, sem_s):
        c = lax.axis_index("c")
        s = lax.axis_index("s")
        zero = jnp.zeros((L,), jnp.float32)

        @pl.loop(0, zr)
        def _zfill(r):
            for cc in range(h // L):
                zbuf[r, pl.ds(cc * L, L)] = zero

        for j in range(nps // zr):
            pltpu.sync_copy(zbuf, acc_sh.at[pl.ds(s * nps + j * zr, zr)])

        @pl.when(s == NS - 1)
        def _ztail():
            pltpu.sync_copy(zbuf.at[pl.ds(0, tail)], acc_sh.at[pl.ds(NS * nps, tail)])

        plsc.subcore_barrier()

        def issue_gathers(k, bb):
            base = s * ec + k * b
            g = s * nchunk + k
            pltpu.sync_copy(src_hbm.at[pl.ds(g, 1)], idx_sv[bb])
            pltpu.sync_copy(dst_hbm.at[pl.ds(g, 1)], idx_dv[bb])
            pltpu.async_copy(tsrc_hbm.at[c].at[idx_sv[bb].at[0]], gsrc[bb], sem_g[bb])
            pltpu.async_copy(tdst_hbm.at[c].at[idx_dv[bb].at[0]], gdst[bb], sem_g[bb])
            pltpu.async_copy(el_hbm.at[c, pl.ds(base, b)], elb[bb], sem_g[bb])

        def wait_gathers(bb):
            pltpu.make_async_copy(
                tsrc_hbm.at[c].at[idx_sv[bb].at[0]], gsrc[bb], sem_g[bb]).wait()
            pltpu.make_async_copy(
                tdst_hbm.at[c].at[idx_dv[bb].at[0]], gdst[bb], sem_g[bb]).wait()
            pltpu.make_async_copy(
                el_hbm.at[c, pl.ds(0, b)], elb[bb], sem_g[bb]).wait()

        def wait_m(bb):
            pltpu.make_async_copy(
                m_buf[bb], m_hbm.at[pl.ds(0, b), pl.ds(c * h, h)], sem_m[bb]
            ).wait()

        def wait_s(bb):
            pltpu.make_async_copy(
                el_hbm.at[c, pl.ds(0, b)], msg[bb], sem_s[bb]).wait()

        issue_gathers(0, 0)

        @pl.loop(0, nchunk // 2)
        def _chunk2(kk):
            for half in range(2):
                k = kk * 2 + half
                other = 1 - half

                if half == 0:
                    @pl.when(kk >= 1)
                    def _wprev():
                        wait_m(other)
                        wait_s(other)
                else:
                    wait_m(other)
                    wait_s(other)

                if half == 0:
                    issue_gathers(k + 1, other)
                else:
                    @pl.when(kk + 1 < nchunk // 2)
                    def _pref():
                        issue_gathers(k + 1, other)

                wait_gathers(half)

                @plsc.parallel_loop(0, b, step=1)
                def _row(r):
                    for cc in range(h // L):
                        off = cc * L
                        es = gsrc[half][r, pl.ds(off, L)]
                        ed = gdst[half][r, pl.ds(off, L)]
                        ev = elb[half][r, pl.ds(off, L)]
                        mv = es + ed + ev
                        m_buf[half][r, pl.ds(off, L)] = mv
                        den = 1.0 + jnp.exp(-mv)
                        fs = gsrc[half][r, pl.ds(h + off, L)]
                        msg[half][r, pl.ds(off, L)] = fs / den

                base = s * ec + k * b
                pltpu.async_copy(
                    m_buf[half], m_hbm.at[pl.ds(base, b), pl.ds(c * h, h)],
                    sem_m[half],
                )
                pltpu.async_copy(
                    msg[half], acc_sh.at[idx_dv[half].at[0]], sem_s[half], add=True
                )

        wait_m(1)
        wait_s(1)
        plsc.subcore_barrier()
        pltpu.sync_copy(acc_sh.at[pl.ds(s * nps, nps)],
                        acc_hbm.at[c, pl.ds(s * nps, nps)])

        @pl.when(s == NS - 1)
        def _otail():
            pltpu.sync_copy(acc_sh.at[pl.ds(NS * nps, tail)],
                            acc_hbm.at[c, pl.ds(NS * nps, tail)])

    src2 = src_idx.reshape(e // b, b)
    dst2 = dst_idx.reshape(e // b, b)
    return edge_kernel(src2, dst2, tsrc, tdst, el)


def _final_call(x, acc, degp, weight, bias):
    n, d = x.shape
    h = d // 2
    nb = n
    grid = 1

    def body(x_ref, acc_ref, degp_ref, w_ref, bb_ref, out_ref):
        a = jnp.concatenate([acc_ref[0], acc_ref[1]], axis=1)
        r = jnp.dot(a, w_ref[...], preferred_element_type=jnp.float32)
        deg_i = jnp.sum(degp_ref[:, 1, :], axis=0)
        inv_i = lax.rsqrt(jnp.maximum(deg_i, 1.0))
        out_ref[...] = x_ref[...] + r * inv_i[:, None] + bb_ref[...]

    return pl.pallas_call(
        body,
        grid=(grid,),
        in_specs=[
            pl.BlockSpec((nb, d), lambda i: (i, 0)),
            pl.BlockSpec((NC, nb, h), lambda i: (0, i, 0)),
            pl.BlockSpec((NW, 2, n), lambda i: (0, 0, 0)),
            pl.BlockSpec((d, d), lambda i: (0, 0)),
            pl.BlockSpec((1, d), lambda i: (0, 0)),
        ],
        out_specs=pl.BlockSpec((nb, d), lambda i: (i, 0)),
        out_shape=jax.ShapeDtypeStruct((n, d), jnp.float32),
    )(x, acc, degp, weight, bias)


def kernel(node_feats, edge_index, edge_feats, W_src, b_src, W_dst, b_dst,
           W_edge, b_edge, weight, bias):
    n, d = node_feats.shape
    e = edge_index.shape[1]
    src_idx = edge_index[0]
    dst_idx = edge_index[1]
    degp = _degree_call(src_idx, dst_idx, n, e)
    tsrc, tdst = _node_tables_call(
        node_feats, W_src, b_src.reshape(1, d), W_dst, b_dst.reshape(1, d), degp
    )
    el = _edge_lin_call(edge_feats, W_edge, b_edge.reshape(1, d))
    m, acc = _edge_gather_scatter_call(src_idx, dst_idx, tsrc, tdst, el, n, e, d)
    rst = _final_call(node_feats, acc, degp, weight, bias.reshape(1, d))
    return (rst, m)


# speedup vs baseline: 2.6745x; 1.0924x over previous
import functools

import jax
import jax.numpy as jnp
from jax import lax
from jax.experimental import pallas as pl
from jax.experimental.pallas import tpu as pltpu
from jax.experimental.pallas import tpu_sc as plsc

NC = 2
NS = 16
NW = NC * NS
L = 16

_SC_PARAMS = pltpu.CompilerParams(
    needs_layout_passes=False, use_tc_tiling_on_sc=False
)


def _degree_call(src_idx, dst_idx, n, e):
    ec = e // NW
    deg_b = 2000
    mesh = plsc.VectorSubcoreMesh(core_axis_name="c", subcore_axis_name="s")

    @functools.partial(
        pl.kernel,
        out_type=jax.ShapeDtypeStruct((NW, 2, n), jnp.float32),
        mesh=mesh,
        compiler_params=_SC_PARAMS,
        scratch_types=[
            pltpu.VMEM((deg_b,), jnp.int32),
            pltpu.VMEM((n,), jnp.float32),
            pltpu.VMEM((n,), jnp.float32),
        ],
    )
    def deg_kernel(src_hbm, dst_hbm, out_hbm, idx_v, dgo_v, dgi_v):
        c = lax.axis_index("c")
        s = lax.axis_index("s")
        w = c * NS + s
        zero = jnp.zeros((L,), jnp.float32)
        ones = jnp.ones((L,), jnp.float32)

        @pl.loop(0, n // L)
        def _zero(i):
            dgo_v[pl.ds(i * L, L)] = zero
            dgi_v[pl.ds(i * L, L)] = zero

        @pl.loop(0, ec // deg_b)
        def _chunk(k):
            base = w * ec + k * deg_b
            pltpu.sync_copy(src_hbm.at[pl.ds(base, deg_b)], idx_v)

            @pl.loop(0, deg_b // L)
            def _src(j):
                iv = idx_v[pl.ds(j * L, L)]
                plsc.addupdate_scatter(dgo_v, [iv], ones)

            pltpu.sync_copy(dst_hbm.at[pl.ds(base, deg_b)], idx_v)

            @pl.loop(0, deg_b // L)
            def _dst(j):
                iv = idx_v[pl.ds(j * L, L)]
                plsc.addupdate_scatter(dgi_v, [iv], ones)

        pltpu.sync_copy(dgo_v, out_hbm.at[w, 0])
        pltpu.sync_copy(dgi_v, out_hbm.at[w, 1])

    return deg_kernel(src_idx, dst_idx)


def _node_tables_call(x, w_src, b_src, w_dst, b_dst, degp):
    n, d = x.shape
    h = d // 2
    nb = n
    grid = 1

    def body(x_ref, ws_ref, bs_ref, wd_ref, bd_ref, degp_ref, tsrc_ref, tdst_ref):
        xv = x_ref[...]
        es = jnp.dot(xv, ws_ref[...], preferred_element_type=jnp.float32) + bs_ref[...]
        deg_o = jnp.sum(degp_ref[:, 0, :], axis=0)
        inv_o = lax.rsqrt(jnp.maximum(deg_o, 1.0))
        fs = xv * inv_o[:, None]
        tsrc_ref[0, :, :h] = es[:, :h]
        tsrc_ref[0, :, h:] = fs[:, :h]
        tsrc_ref[1, :, :h] = es[:, h:]
        tsrc_ref[1, :, h:] = fs[:, h:]
        ed = jnp.dot(xv, wd_ref[...], preferred_element_type=jnp.float32) + bd_ref[...]
        tdst_ref[0] = ed[:, :h]
        tdst_ref[1] = ed[:, h:]

    return pl.pallas_call(
        body,
        grid=(grid,),
        in_specs=[
            pl.BlockSpec((nb, d), lambda i: (i, 0)),
            pl.BlockSpec((d, d), lambda i: (0, 0)),
            pl.BlockSpec((1, d), lambda i: (0, 0)),
            pl.BlockSpec((d, d), lambda i: (0, 0)),
            pl.BlockSpec((1, d), lambda i: (0, 0)),
            pl.BlockSpec((NW, 2, n), lambda i: (0, 0, 0)),
        ],
        out_specs=[
            pl.BlockSpec((NC, nb, d), lambda i: (0, i, 0)),
            pl.BlockSpec((NC, nb, h), lambda i: (0, i, 0)),
        ],
        out_shape=[
            jax.ShapeDtypeStruct((NC, n, d), jnp.float32),
            jax.ShapeDtypeStruct((NC, n, h), jnp.float32),
        ],
    )(x, w_src, b_src, w_dst, b_dst, degp)


def _edge_lin_call(edge_feats, w_edge, b_edge):
    e, d = edge_feats.shape
    h = d // 2
    be = 2000
    grid = e // be

    def body(ef_ref, we_ref, bb_ref, out_ref):
        lin = (
            jnp.dot(ef_ref[...], we_ref[...], preferred_element_type=jnp.float32)
            + bb_ref[...]
        )
        out_ref[0] = lin[:, :h]
        out_ref[1] = lin[:, h:]

    return pl.pallas_call(
        body,
        grid=(grid,),
        in_specs=[
            pl.BlockSpec((be, d), lambda i: (i, 0)),
            pl.BlockSpec((d, d), lambda i: (0, 0)),
            pl.BlockSpec((1, d), lambda i: (0, 0)),
        ],
        out_specs=pl.BlockSpec((NC, be, h), lambda i: (0, i, 0)),
        out_shape=jax.ShapeDtypeStruct((NC, e, h), jnp.float32),
    )(edge_feats, w_edge, b_edge)


def _edge_gather_scatter_call(src_idx, dst_idx, tsrc, tdst, el, n, e, d):
    h = d // 2
    ec = e // NS
    b = 80
    nchunk = ec // b
    nps = 624
    tail = n - NS * nps
    zr = 208
    mesh = plsc.VectorSubcoreMesh(core_axis_name="c", subcore_axis_name="s")

    @functools.partial(
        pl.kernel,
        out_type=(
            jax.ShapeDtypeStruct((e, d), jnp.float32),
            jax.ShapeDtypeStruct((NC, n, h), jnp.float32),
        ),
        mesh=mesh,
        compiler_params=_SC_PARAMS,
        scratch_types=[
            [pltpu.VMEM((1, b), jnp.int32)] * 2,
            [pltpu.VMEM((1, b), jnp.int32)] * 2,
            [pltpu.VMEM((b, d), jnp.float32)] * 2,
            [pltpu.VMEM((b, h), jnp.float32)] * 2,
            [pltpu.VMEM((b, h), jnp.float32)] * 2,
            [pltpu.VMEM((b, h), jnp.float32)] * 2,
            [pltpu.VMEM((b, h), jnp.float32)] * 2,
            pltpu.VMEM((zr, h), jnp.float32),
            pltpu.VMEM_SHARED((n, h), jnp.float32),
            [pltpu.SemaphoreType.DMA] * 2,
            [pltpu.SemaphoreType.DMA] * 2,
            [pltpu.SemaphoreType.DMA] * 2,
            [pltpu.SemaphoreType.DMA] * 2,
        ],
    )
    def edge_kernel(src_hbm, dst_hbm, tsrc_hbm, tdst_hbm, el_hbm, m_hbm, acc_hbm,
                    idx_sv, idx_dv, gsrc, gdst, elb, m_buf, msg, zbuf, acc_sh,
                    sem_g, sem_m, sem_s, sem_i):
        c = lax.axis_index("c")
        s = lax.axis_index("s")
        zero = jnp.zeros((L,), jnp.float32)

        @pl.loop(0, zr)
        def _zfill(r):
            for cc in range(h // L):
                zbuf[r, pl.ds(cc * L, L)] = zero

        for j in range(nps // zr):
            pltpu.sync_copy(zbuf, acc_sh.at[pl.ds(s * nps + j * zr, zr)])

        @pl.when(s == NS - 1)
        def _ztail():
            pltpu.sync_copy(zbuf.at[pl.ds(0, tail)], acc_sh.at[pl.ds(NS * nps, tail)])

        plsc.subcore_barrier()

        def issue_gathers(k, bb):
            base = s * ec + k * b
            g = s * nchunk + k
            pltpu.async_copy(src_hbm.at[pl.ds(g, 1)], idx_sv[bb], sem_i[bb])
            pltpu.async_copy(dst_hbm.at[pl.ds(g, 1)], idx_dv[bb], sem_i[bb])
            pltpu.make_async_copy(
                src_hbm.at[pl.ds(0, 1)], idx_sv[bb], sem_i[bb]).wait()
            pltpu.make_async_copy(
                dst_hbm.at[pl.ds(0, 1)], idx_dv[bb], sem_i[bb]).wait()
            pltpu.async_copy(tsrc_hbm.at[c].at[idx_sv[bb].at[0]], gsrc[bb], sem_g[bb])
            pltpu.async_copy(tdst_hbm.at[c].at[idx_dv[bb].at[0]], gdst[bb], sem_g[bb])
            pltpu.async_copy(el_hbm.at[c, pl.ds(base, b)], elb[bb], sem_g[bb])

        def wait_gathers(bb):
            pltpu.make_async_copy(
                tsrc_hbm.at[c].at[idx_sv[bb].at[0]], gsrc[bb], sem_g[bb]).wait()
            pltpu.make_async_copy(
                tdst_hbm.at[c].at[idx_dv[bb].at[0]], gdst[bb], sem_g[bb]).wait()
            pltpu.make_async_copy(
                el_hbm.at[c, pl.ds(0, b)], elb[bb], sem_g[bb]).wait()

        def wait_m(bb):
            pltpu.make_async_copy(
                m_buf[bb], m_hbm.at[pl.ds(0, b), pl.ds(c * h, h)], sem_m[bb]
            ).wait()

        def wait_s(bb):
            pltpu.make_async_copy(
                el_hbm.at[c, pl.ds(0, b)], msg[bb], sem_s[bb]).wait()

        issue_gathers(0, 0)

        @pl.loop(0, nchunk // 2)
        def _chunk2(kk):
            for half in range(2):
                k = kk * 2 + half
                other = 1 - half

                if half == 0:
                    @pl.when(kk >= 1)
                    def _wprev():
                        wait_m(other)
                        wait_s(other)
                else:
                    wait_m(other)
                    wait_s(other)

                if half == 0:
                    issue_gathers(k + 1, other)
                else:
                    @pl.when(kk + 1 < nchunk // 2)
                    def _pref():
                        issue_gathers(k + 1, other)

                wait_gathers(half)

                @plsc.parallel_loop(0, b, step=1)
                def _row(r):
                    for cc in range(h // L):
                        off = cc * L
                        es = gsrc[half][r, pl.ds(off, L)]
                        ed = gdst[half][r, pl.ds(off, L)]
                        ev = elb[half][r, pl.ds(off, L)]
                        mv = es + ed + ev
                        m_buf[half][r, pl.ds(off, L)] = mv
                        den = 1.0 + jnp.exp(-mv)
                        fs = gsrc[half][r, pl.ds(h + off, L)]
                        msg[half][r, pl.ds(off, L)] = fs / den

                base = s * ec + k * b
                pltpu.async_copy(
                    m_buf[half], m_hbm.at[pl.ds(base, b), pl.ds(c * h, h)],
                    sem_m[half],
                )
                pltpu.async_copy(
                    msg[half], acc_sh.at[idx_dv[half].at[0]], sem_s[half], add=True
                )

        wait_m(1)
        wait_s(1)
        plsc.subcore_barrier()
        pltpu.sync_copy(acc_sh.at[pl.ds(s * nps, nps)],
                        acc_hbm.at[c, pl.ds(s * nps, nps)])

        @pl.when(s == NS - 1)
        def _otail():
            pltpu.sync_copy(acc_sh.at[pl.ds(NS * nps, tail)],
                            acc_hbm.at[c, pl.ds(NS * nps, tail)])

    src2 = src_idx.reshape(e // b, b)
    dst2 = dst_idx.reshape(e // b, b)
    return edge_kernel(src2, dst2, tsrc, tdst, el)


def _final_call(x, acc, degp, weight, bias):
    n, d = x.shape
    h = d // 2
    nb = n
    grid = 1

    def body(x_ref, acc_ref, degp_ref, w_ref, bb_ref, out_ref):
        a = jnp.concatenate([acc_ref[0], acc_ref[1]], axis=1)
        r = jnp.dot(a, w_ref[...], preferred_element_type=jnp.float32)
        deg_i = jnp.sum(degp_ref[:, 1, :], axis=0)
        inv_i = lax.rsqrt(jnp.maximum(deg_i, 1.0))
        out_ref[...] = x_ref[...] + r * inv_i[:, None] + bb_ref[...]

    return pl.pallas_call(
        body,
        grid=(grid,),
        in_specs=[
            pl.BlockSpec((nb, d), lambda i: (i, 0)),
            pl.BlockSpec((NC, nb, h), lambda i: (0, i, 0)),
            pl.BlockSpec((NW, 2, n), lambda i: (0, 0, 0)),
            pl.BlockSpec((d, d), lambda i: (0, 0)),
            pl.BlockSpec((1, d), lambda i: (0, 0)),
        ],
        out_specs=pl.BlockSpec((nb, d), lambda i: (i, 0)),
        out_shape=jax.ShapeDtypeStruct((n, d), jnp.float32),
    )(x, acc, degp, weight, bias)


def kernel(node_feats, edge_index, edge_feats, W_src, b_src, W_dst, b_dst,
           W_edge, b_edge, weight, bias):
    n, d = node_feats.shape
    e = edge_index.shape[1]
    src_idx = edge_index[0]
    dst_idx = edge_index[1]
    degp = _degree_call(src_idx, dst_idx, n, e)
    tsrc, tdst = _node_tables_call(
        node_feats, W_src, b_src.reshape(1, d), W_dst, b_dst.reshape(1, d), degp
    )
    el = _edge_lin_call(edge_feats, W_edge, b_edge.reshape(1, d))
    m, acc = _edge_gather_scatter_call(src_idx, dst_idx, tsrc, tdst, el, n, e, d)
    rst = _final_call(node_feats, acc, degp, weight, bias.reshape(1, d))
    return (rst, m)


# speedup vs baseline: 2.8464x; 1.0643x over previous
import functools

import jax
import jax.numpy as jnp
from jax import lax
from jax.experimental import pallas as pl
from jax.experimental.pallas import tpu as pltpu
from jax.experimental.pallas import tpu_sc as plsc

NC = 2
NS = 16
NW = NC * NS
L = 16

_SC_PARAMS = pltpu.CompilerParams(
    needs_layout_passes=False, use_tc_tiling_on_sc=False
)


def _degree_call(src_idx, dst_idx, n, e):
    ec = e // NW
    deg_b = 2000
    mesh = plsc.VectorSubcoreMesh(core_axis_name="c", subcore_axis_name="s")

    @functools.partial(
        pl.kernel,
        out_type=jax.ShapeDtypeStruct((NW, 2, n), jnp.float32),
        mesh=mesh,
        compiler_params=_SC_PARAMS,
        scratch_types=[
            pltpu.VMEM((deg_b,), jnp.int32),
            pltpu.VMEM((n,), jnp.float32),
            pltpu.VMEM((n,), jnp.float32),
        ],
    )
    def deg_kernel(src_hbm, dst_hbm, out_hbm, idx_v, dgo_v, dgi_v):
        c = lax.axis_index("c")
        s = lax.axis_index("s")
        w = c * NS + s
        zero = jnp.zeros((L,), jnp.float32)
        ones = jnp.ones((L,), jnp.float32)

        @pl.loop(0, n // L)
        def _zero(i):
            dgo_v[pl.ds(i * L, L)] = zero
            dgi_v[pl.ds(i * L, L)] = zero

        @pl.loop(0, ec // deg_b)
        def _chunk(k):
            base = w * ec + k * deg_b
            pltpu.sync_copy(src_hbm.at[pl.ds(base, deg_b)], idx_v)

            @pl.loop(0, deg_b // L)
            def _src(j):
                iv = idx_v[pl.ds(j * L, L)]
                plsc.addupdate_scatter(dgo_v, [iv], ones)

            pltpu.sync_copy(dst_hbm.at[pl.ds(base, deg_b)], idx_v)

            @pl.loop(0, deg_b // L)
            def _dst(j):
                iv = idx_v[pl.ds(j * L, L)]
                plsc.addupdate_scatter(dgi_v, [iv], ones)

        pltpu.sync_copy(dgo_v, out_hbm.at[w, 0])
        pltpu.sync_copy(dgi_v, out_hbm.at[w, 1])

    return deg_kernel(src_idx, dst_idx)


def _node_tables_call(x, w_src, b_src, w_dst, b_dst, degp):
    n, d = x.shape
    h = d // 2
    nb = n
    grid = 1

    def body(x_ref, ws_ref, bs_ref, wd_ref, bd_ref, degp_ref, tsrc_ref, tdst_ref):
        xv = x_ref[...]
        es = jnp.dot(xv, ws_ref[...], preferred_element_type=jnp.float32) + bs_ref[...]
        deg_o = jnp.sum(degp_ref[:, 0, :], axis=0)
        inv_o = lax.rsqrt(jnp.maximum(deg_o, 1.0))
        fs = xv * inv_o[:, None]
        tsrc_ref[0, :, :h] = es[:, :h]
        tsrc_ref[0, :, h:] = fs[:, :h]
        tsrc_ref[1, :, :h] = es[:, h:]
        tsrc_ref[1, :, h:] = fs[:, h:]
        ed = jnp.dot(xv, wd_ref[...], preferred_element_type=jnp.float32) + bd_ref[...]
        tdst_ref[0] = ed[:, :h]
        tdst_ref[1] = ed[:, h:]

    return pl.pallas_call(
        body,
        grid=(grid,),
        in_specs=[
            pl.BlockSpec((nb, d), lambda i: (i, 0)),
            pl.BlockSpec((d, d), lambda i: (0, 0)),
            pl.BlockSpec((1, d), lambda i: (0, 0)),
            pl.BlockSpec((d, d), lambda i: (0, 0)),
            pl.BlockSpec((1, d), lambda i: (0, 0)),
            pl.BlockSpec((NW, 2, n), lambda i: (0, 0, 0)),
        ],
        out_specs=[
            pl.BlockSpec((NC, nb, d), lambda i: (0, i, 0)),
            pl.BlockSpec((NC, nb, h), lambda i: (0, i, 0)),
        ],
        out_shape=[
            jax.ShapeDtypeStruct((NC, n, d), jnp.float32),
            jax.ShapeDtypeStruct((NC, n, h), jnp.float32),
        ],
    )(x, w_src, b_src, w_dst, b_dst, degp)


def _edge_lin_call(edge_feats, w_edge, b_edge):
    e, d = edge_feats.shape
    h = d // 2
    be = 8000
    grid = e // be

    def body(ef_ref, we_ref, bb_ref, out_ref):
        lin = (
            jnp.dot(ef_ref[...], we_ref[...], preferred_element_type=jnp.float32)
            + bb_ref[...]
        )
        out_ref[0] = lin[:, :h]
        out_ref[1] = lin[:, h:]

    return pl.pallas_call(
        body,
        grid=(grid,),
        in_specs=[
            pl.BlockSpec((be, d), lambda i: (i, 0)),
            pl.BlockSpec((d, d), lambda i: (0, 0)),
            pl.BlockSpec((1, d), lambda i: (0, 0)),
        ],
        out_specs=pl.BlockSpec((NC, be, h), lambda i: (0, i, 0)),
        out_shape=jax.ShapeDtypeStruct((NC, e, h), jnp.float32),
    )(edge_feats, w_edge, b_edge)


def _edge_gather_scatter_call(src_idx, dst_idx, tsrc, tdst, el, n, e, d):
    h = d // 2
    ec = e // NS
    b = 80
    nchunk = ec // b
    nps = 624
    tail = n - NS * nps
    zr = 208
    mesh = plsc.VectorSubcoreMesh(core_axis_name="c", subcore_axis_name="s")

    @functools.partial(
        pl.kernel,
        out_type=(
            jax.ShapeDtypeStruct((e, d), jnp.float32),
            jax.ShapeDtypeStruct((NC, n, h), jnp.float32),
        ),
        mesh=mesh,
        compiler_params=_SC_PARAMS,
        scratch_types=[
            [pltpu.VMEM((1, b), jnp.int32)] * 2,
            [pltpu.VMEM((1, b), jnp.int32)] * 2,
            [pltpu.VMEM((b, d), jnp.float32)] * 2,
            [pltpu.VMEM((b, h), jnp.float32)] * 2,
            [pltpu.VMEM((b, h), jnp.float32)] * 2,
            [pltpu.VMEM((b, h), jnp.float32)] * 2,
            [pltpu.VMEM((b, h), jnp.float32)] * 2,
            pltpu.VMEM((zr, h), jnp.float32),
            pltpu.VMEM_SHARED((n, h), jnp.float32),
            [pltpu.SemaphoreType.DMA] * 2,
            [pltpu.SemaphoreType.DMA] * 2,
            [pltpu.SemaphoreType.DMA] * 2,
            [pltpu.SemaphoreType.DMA] * 2,
        ],
    )
    def edge_kernel(src_hbm, dst_hbm, tsrc_hbm, tdst_hbm, el_hbm, m_hbm, acc_hbm,
                    idx_sv, idx_dv, gsrc, gdst, elb, m_buf, msg, zbuf, acc_sh,
                    sem_g, sem_m, sem_s, sem_i):
        c = lax.axis_index("c")
        s = lax.axis_index("s")
        zero = jnp.zeros((L,), jnp.float32)

        @pl.loop(0, zr)
        def _zfill(r):
            for cc in range(h // L):
                zbuf[r, pl.ds(cc * L, L)] = zero

        for j in range(nps // zr):
            pltpu.sync_copy(zbuf, acc_sh.at[pl.ds(s * nps + j * zr, zr)])

        @pl.when(s == NS - 1)
        def _ztail():
            pltpu.sync_copy(zbuf.at[pl.ds(0, tail)], acc_sh.at[pl.ds(NS * nps, tail)])

        plsc.subcore_barrier()

        def issue_gathers(k, bb):
            base = s * ec + k * b
            g = s * nchunk + k
            pltpu.async_copy(src_hbm.at[pl.ds(g, 1)], idx_sv[bb], sem_i[bb])
            pltpu.async_copy(dst_hbm.at[pl.ds(g, 1)], idx_dv[bb], sem_i[bb])
            pltpu.make_async_copy(
                src_hbm.at[pl.ds(0, 1)], idx_sv[bb], sem_i[bb]).wait()
            pltpu.make_async_copy(
                dst_hbm.at[pl.ds(0, 1)], idx_dv[bb], sem_i[bb]).wait()
            pltpu.async_copy(tsrc_hbm.at[c].at[idx_sv[bb].at[0]], gsrc[bb], sem_g[bb])
            pltpu.async_copy(tdst_hbm.at[c].at[idx_dv[bb].at[0]], gdst[bb], sem_g[bb])
            pltpu.async_copy(el_hbm.at[c, pl.ds(base, b)], elb[bb], sem_g[bb])

        def wait_gathers(bb):
            pltpu.make_async_copy(
                tsrc_hbm.at[c].at[idx_sv[bb].at[0]], gsrc[bb], sem_g[bb]).wait()
            pltpu.make_async_copy(
                tdst_hbm.at[c].at[idx_dv[bb].at[0]], gdst[bb], sem_g[bb]).wait()
            pltpu.make_async_copy(
                el_hbm.at[c, pl.ds(0, b)], elb[bb], sem_g[bb]).wait()

        def wait_m(bb):
            pltpu.make_async_copy(
                m_buf[bb], m_hbm.at[pl.ds(0, b), pl.ds(c * h, h)], sem_m[bb]
            ).wait()

        def wait_s(bb):
            pltpu.make_async_copy(
                el_hbm.at[c, pl.ds(0, b)], msg[bb], sem_s[bb]).wait()

        issue_gathers(0, 0)

        @pl.loop(0, nchunk // 2)
        def _chunk2(kk):
            for half in range(2):
                k = kk * 2 + half
                other = 1 - half

                if half == 0:
                    @pl.when(kk >= 1)
                    def _wprev():
                        wait_m(other)
                        wait_s(other)
                else:
                    wait_m(other)
                    wait_s(other)

                if half == 0:
                    issue_gathers(k + 1, other)
                else:
                    @pl.when(kk + 1 < nchunk // 2)
                    def _pref():
                        issue_gathers(k + 1, other)

                wait_gathers(half)

                @plsc.parallel_loop(0, b, step=1)
                def _row(r):
                    for cc in range(h // L):
                        off = cc * L
                        es = gsrc[half][r, pl.ds(off, L)]
                        ed = gdst[half][r, pl.ds(off, L)]
                        ev = elb[half][r, pl.ds(off, L)]
                        mv = es + ed + ev
                        m_buf[half][r, pl.ds(off, L)] = mv
                        den = 1.0 + jnp.exp(-mv)
                        fs = gsrc[half][r, pl.ds(h + off, L)]
                        msg[half][r, pl.ds(off, L)] = fs / den

                base = s * ec + k * b
                pltpu.async_copy(
                    m_buf[half], m_hbm.at[pl.ds(base, b), pl.ds(c * h, h)],
                    sem_m[half],
                )
                pltpu.async_copy(
                    msg[half], acc_sh.at[idx_dv[half].at[0]], sem_s[half], add=True
                )

        wait_m(1)
        wait_s(1)
        plsc.subcore_barrier()
        pltpu.sync_copy(acc_sh.at[pl.ds(s * nps, nps)],
                        acc_hbm.at[c, pl.ds(s * nps, nps)])

        @pl.when(s == NS - 1)
        def _otail():
            pltpu.sync_copy(acc_sh.at[pl.ds(NS * nps, tail)],
                            acc_hbm.at[c, pl.ds(NS * nps, tail)])

    src2 = src_idx.reshape(e // b, b)
    dst2 = dst_idx.reshape(e // b, b)
    return edge_kernel(src2, dst2, tsrc, tdst, el)


def _final_call(x, acc, degp, weight, bias):
    n, d = x.shape
    h = d // 2
    nb = n
    grid = 1

    def body(x_ref, acc_ref, degp_ref, w_ref, bb_ref, out_ref):
        a = jnp.concatenate([acc_ref[0], acc_ref[1]], axis=1)
        r = jnp.dot(a, w_ref[...], preferred_element_type=jnp.float32)
        deg_i = jnp.sum(degp_ref[:, 1, :], axis=0)
        inv_i = lax.rsqrt(jnp.maximum(deg_i, 1.0))
        out_ref[...] = x_ref[...] + r * inv_i[:, None] + bb_ref[...]

    return pl.pallas_call(
        body,
        grid=(grid,),
        in_specs=[
            pl.BlockSpec((nb, d), lambda i: (i, 0)),
            pl.BlockSpec((NC, nb, h), lambda i: (0, i, 0)),
            pl.BlockSpec((NW, 2, n), lambda i: (0, 0, 0)),
            pl.BlockSpec((d, d), lambda i: (0, 0)),
            pl.BlockSpec((1, d), lambda i: (0, 0)),
        ],
        out_specs=pl.BlockSpec((nb, d), lambda i: (i, 0)),
        out_shape=jax.ShapeDtypeStruct((n, d), jnp.float32),
    )(x, acc, degp, weight, bias)


def kernel(node_feats, edge_index, edge_feats, W_src, b_src, W_dst, b_dst,
           W_edge, b_edge, weight, bias):
    n, d = node_feats.shape
    e = edge_index.shape[1]
    src_idx = edge_index[0]
    dst_idx = edge_index[1]
    degp = _degree_call(src_idx, dst_idx, n, e)
    tsrc, tdst = _node_tables_call(
        node_feats, W_src, b_src.reshape(1, d), W_dst, b_dst.reshape(1, d), degp
    )
    el = _edge_lin_call(edge_feats, W_edge, b_edge.reshape(1, d))
    m, acc = _edge_gather_scatter_call(src_idx, dst_idx, tsrc, tdst, el, n, e, d)
    rst = _final_call(node_feats, acc, degp, weight, bias.reshape(1, d))
    return (rst, m)
